# baseline (speedup 1.0000x reference)
import functools

import jax
import jax.numpy as jnp
from jax import lax
from jax.experimental import pallas as pl
from jax.experimental.pallas import tpu as pltpu
from jax.experimental.pallas import tpu_sc as plsc

DIM = 64
BATCH = 4096
SEQ = 200
NUM_CORES = 2
NUM_SUBCORES = 16
NUM_WORKERS = NUM_CORES * NUM_SUBCORES
BBLK = BATCH // NUM_WORKERS
NBUF = 4
NGROUP = SEQ // NBUF
LANES = 16
MGRP = DIM // LANES


def _build():
    mesh = plsc.VectorSubcoreMesh(core_axis_name="c", subcore_axis_name="s")

    @functools.partial(
        pl.kernel,
        mesh=mesh,
        out_type=jax.ShapeDtypeStruct(
            (SEQ, DIM // 8, NUM_WORKERS, 8 * BBLK), jnp.float32),
        scratch_types=[
            pltpu.VMEM((SEQ, BBLK), jnp.int32),
            pltpu.VMEM((NBUF, BBLK, DIM // 2), jnp.float32),
            pltpu.VMEM((NBUF, BBLK, DIM // 2), jnp.float32),
            pltpu.VMEM((NBUF, DIM // 8, 8 * BBLK), jnp.float32),
            pltpu.SemaphoreType.DMA((NBUF,)),
            pltpu.SemaphoreType.DMA((NBUF,)),
        ],
        compiler_params=pltpu.CompilerParams(
            use_tc_tiling_on_sc=False, needs_layout_passes=False),
    )
    def emb(idxt_hbm, wlo_hbm, whi_hbm, out_hbm,
            idx_v, rlo_v, rhi_v, obuf_v, gsem, osem):
        wid = lax.axis_index("s") * NUM_CORES + lax.axis_index("c")
        pltpu.sync_copy(idxt_hbm.at[:, pl.ds(wid * BBLK, BBLK)], idx_v)

        lane = lax.iota(jnp.int32, LANES)
        jrow = [k * LANES + lane for k in range(BBLK // LANES)]

        def fire_gather(s, b):
            idx = idx_v.at[s]
            pltpu.async_copy(wlo_hbm.at[idx], rlo_v.at[b], gsem.at[b])
            pltpu.async_copy(whi_hbm.at[idx], rhi_v.at[b], gsem.at[b])

        def drain_gather(s, b):
            idx = idx_v.at[s]
            pltpu.make_async_copy(wlo_hbm.at[idx], rlo_v.at[b], gsem.at[b]).wait()
            pltpu.make_async_copy(whi_hbm.at[idx], rhi_v.at[b], gsem.at[b]).wait()

        HDIM = DIM // 2

        def transpose(b):
            @plsc.parallel_loop(0, HDIM, 1, unroll=2)
            def _(d0):
                dp = lax.bitwise_and(d0 + lane, HDIM - 1)
                drow = lax.shift_right_logical(dp, 3)
                dcol = lax.shift_left(lax.bitwise_and(dp, 7), 7)
                for k in range(BBLK // LANES):
                    v = plsc.load_gather(rlo_v.at[b], [jrow[k], dp])
                    plsc.store_scatter(
                        obuf_v.at[b], [drow, dcol + jrow[k]], v)
                    w = plsc.load_gather(rhi_v.at[b], [jrow[k], dp])
                    plsc.store_scatter(
                        obuf_v.at[b], [drow + HDIM // 8, dcol + jrow[k]], w)

        def out_slice(s):
            return out_hbm.at[s, :, wid]

        def fire_out(s, b):
            pltpu.async_copy(obuf_v.at[b], out_slice(s), osem.at[b])

        def drain_out(s, b):
            pltpu.make_async_copy(obuf_v.at[b], out_slice(s), osem.at[b]).wait()

        for b in range(2):
            fire_gather(b, b)

        def body(g, carry):
            s0 = g * NBUF
            for b in range(NBUF):
                s = s0 + b
                b2 = (b + 2) % NBUF

                @pl.when(s + 2 < SEQ)
                def _():
                    @pl.when(s - 2 >= 0)
                    def _():
                        drain_out(s - 2, b2)
                    fire_gather(s + 2, b2)

                drain_gather(s, b)
                transpose(b)
                fire_out(s, b)
            return carry

        lax.fori_loop(0, NGROUP, body, 0)

        for s in range(SEQ - 4, SEQ):
            drain_out(s, s % NBUF)

    return emb


_emb = _build()


def kernel(token_ids, weight):
    out4 = _emb(token_ids.T, weight[:, :DIM // 2], weight[:, DIM // 2:])
    out5 = out4.reshape(SEQ, DIM // 8, NUM_WORKERS, 8, BBLK)
    return out5.transpose(2, 4, 0, 1, 3).reshape(BATCH, SEQ, DIM)


# speedup vs baseline: 2.0338x; 2.0338x over previous
import functools

import jax
import jax.numpy as jnp
from jax import lax
from jax.experimental import pallas as pl
from jax.experimental.pallas import tpu as pltpu
from jax.experimental.pallas import tpu_sc as plsc

DIM = 64
BATCH = 4096
SEQ = 200
NUM_CORES = 2
NUM_SUBCORES = 16
NUM_WORKERS = NUM_CORES * NUM_SUBCORES
BBLK = BATCH // NUM_WORKERS
NBUF = 5
NGROUP = SEQ // NBUF
LANES = 16
MGRP = DIM // LANES


def _build():
    mesh = plsc.VectorSubcoreMesh(core_axis_name="c", subcore_axis_name="s")

    @functools.partial(
        pl.kernel,
        mesh=mesh,
        out_type=jax.ShapeDtypeStruct(
            (SEQ, DIM // 8, NUM_WORKERS, 8 * BBLK), jnp.float32),
        scratch_types=[
            pltpu.VMEM((SEQ // 8, 8, BBLK), jnp.int32),
            pltpu.VMEM((NBUF, BBLK, DIM), jnp.float32),
            pltpu.VMEM((NBUF, DIM // 8, 8 * BBLK), jnp.float32),
            pltpu.SemaphoreType.DMA((NBUF,)),
            pltpu.SemaphoreType.DMA((NBUF,)),
        ],
        compiler_params=pltpu.CompilerParams(
            use_tc_tiling_on_sc=False, needs_layout_passes=False),
    )
    def emb(idxt_hbm, w_hbm, out_hbm, idx_v, rows_v, obuf_v, gsem, osem):
        wid = lax.axis_index("s") * NUM_CORES + lax.axis_index("c")
        pltpu.sync_copy(idxt_hbm.at[:, wid], idx_v)

        lane = lax.iota(jnp.int32, LANES)
        jrow = [k * LANES + lane for k in range(BBLK // LANES)]

        def idx_row(s):
            return idx_v.at[lax.shift_right_logical(s, 3),
                            lax.bitwise_and(s, 7)]

        def fire_gather(s, b):
            pltpu.async_copy(w_hbm.at[idx_row(s)], rows_v.at[b], gsem.at[b])

        def drain_gather(s, b):
            pltpu.make_async_copy(
                w_hbm.at[idx_row(s)], rows_v.at[b], gsem.at[b]).wait()

        def transpose(b):
            @plsc.parallel_loop(0, DIM, 1, unroll=2)
            def _(d):
                dp = lax.bitwise_and(d + lane, DIM - 1)
                drow = lax.shift_right_logical(dp, 3)
                dcol = lax.shift_left(lax.bitwise_and(dp, 7), 7)
                for k in range(BBLK // LANES):
                    v = plsc.load_gather(rows_v.at[b], [jrow[k], dp])
                    plsc.store_scatter(
                        obuf_v.at[b], [drow, dcol + jrow[k]], v)

        def out_slice(s):
            return out_hbm.at[s, :, wid]

        def fire_out(s, b):
            pltpu.async_copy(obuf_v.at[b], out_slice(s), osem.at[b])

        def drain_out(s, b):
            pltpu.make_async_copy(obuf_v.at[b], out_slice(s), osem.at[b]).wait()

        for b in range(2):
            fire_gather(b, b)

        def body(g, carry):
            s0 = g * NBUF
            for b in range(NBUF):
                s = s0 + b
                b2 = (b + 2) % NBUF

                @pl.when(s + 2 < SEQ)
                def _():
                    @pl.when(s + 2 - NBUF >= 0)
                    def _():
                        drain_out(s + 2 - NBUF, b2)
                    fire_gather(s + 2, b2)

                drain_gather(s, b)
                transpose(b)
                fire_out(s, b)
            return carry

        lax.fori_loop(0, NGROUP, body, 0)

        for s in range(SEQ - NBUF, SEQ):
            drain_out(s, s % NBUF)

    return emb


_emb = _build()


def kernel(token_ids, weight):
    idxt = token_ids.T.reshape(SEQ // 8, 8, NUM_WORKERS, BBLK).transpose(
        0, 2, 1, 3)
    out4 = _emb(idxt, weight)
    out5 = out4.reshape(SEQ, DIM // 8, NUM_WORKERS, 8, BBLK)
    return out5.transpose(2, 4, 0, 1, 3).reshape(BATCH, SEQ, DIM)


# speedup vs baseline: 2.0358x; 1.0010x over previous
import functools

import jax
import jax.numpy as jnp
from jax import lax
from jax.experimental import pallas as pl
from jax.experimental.pallas import tpu as pltpu
from jax.experimental.pallas import tpu_sc as plsc

DIM = 64
BATCH = 4096
SEQ = 200
NUM_CORES = 2
NUM_SUBCORES = 16
NUM_WORKERS = NUM_CORES * NUM_SUBCORES
BBLK = BATCH // NUM_WORKERS
NBUF = 5
NGROUP = SEQ // NBUF
LANES = 16


def _build():
    mesh = plsc.VectorSubcoreMesh(core_axis_name="c", subcore_axis_name="s")

    @functools.partial(
        pl.kernel,
        mesh=mesh,
        out_type=jax.ShapeDtypeStruct(
            (SEQ, DIM // 8, NUM_WORKERS, 8 * BBLK), jnp.float32),
        scratch_types=[
            pltpu.VMEM((SEQ // 8, 8, BBLK), jnp.int32),
            pltpu.VMEM((NBUF, BBLK, DIM), jnp.float32),
            pltpu.VMEM((NBUF, DIM // 8, 8 * BBLK), jnp.float32),
            pltpu.SemaphoreType.DMA((NBUF,)),
            pltpu.SemaphoreType.DMA((NBUF,)),
        ],
        compiler_params=pltpu.CompilerParams(
            use_tc_tiling_on_sc=False, needs_layout_passes=False),
    )
    def emb(idxt_hbm, w_hbm, out_hbm, idx_v, rows_v, obuf_v, gsem, osem):
        wid = lax.axis_index("s") * NUM_CORES + lax.axis_index("c")
        pltpu.sync_copy(idxt_hbm.at[:, wid], idx_v)

        lane = lax.iota(jnp.int32, LANES)
        jrow = [k * LANES + lane for k in range(BBLK // LANES)]

        def idx_row(s):
            return idx_v.at[lax.shift_right_logical(s, 3),
                            lax.bitwise_and(s, 7)]

        def fire_gather(s, b):
            pltpu.async_copy(w_hbm.at[idx_row(s)], rows_v.at[b], gsem.at[b])

        def drain_gather(s, b):
            pltpu.make_async_copy(
                w_hbm.at[idx_row(s)], rows_v.at[b], gsem.at[b]).wait()

        def transpose(b):
            @plsc.parallel_loop(0, DIM, 1, unroll=2)
            def _(d):
                dp = lax.bitwise_and(d + lane, DIM - 1)
                drow = lax.shift_right_logical(dp, 3)
                dcol = lax.shift_left(lax.bitwise_and(dp, 7), 7)
                for k in range(BBLK // LANES):
                    v = plsc.load_gather(rows_v.at[b], [jrow[k], dp])
                    plsc.store_scatter(
                        obuf_v.at[b], [drow, dcol + jrow[k]], v)

        def out_slice(s):
            return out_hbm.at[s, :, wid]

        def fire_out(s, b):
            pltpu.async_copy(obuf_v.at[b], out_slice(s), osem.at[b])

        def drain_out(s, b):
            pltpu.make_async_copy(obuf_v.at[b], out_slice(s), osem.at[b]).wait()

        for b in range(2):
            fire_gather(b, b)

        def body(g, carry):
            s0 = g * NBUF
            for b in range(NBUF):
                s = s0 + b
                b2 = (b + 2) % NBUF

                @pl.when(s + 2 < SEQ)
                def _():
                    @pl.when(s + 2 - NBUF >= 0)
                    def _():
                        drain_out(s + 2 - NBUF, b2)
                    fire_gather(s + 2, b2)

                drain_gather(s, b)
                transpose(b)
                fire_out(s, b)
            return carry

        lax.fori_loop(0, NGROUP, body, 0)

        for s in range(SEQ - NBUF, SEQ):
            drain_out(s, s % NBUF)

    return emb


_emb = _build()


def kernel(token_ids, weight):
    idxt = token_ids.T.reshape(SEQ // 8, 8, NUM_WORKERS, BBLK).transpose(
        0, 2, 1, 3)
    out4 = _emb(idxt, weight)
    out5 = out4.reshape(SEQ, DIM // 8, NUM_WORKERS, 8, BBLK)
    return out5.transpose(2, 4, 0, 1, 3).reshape(BATCH, SEQ, DIM)
